# baseline (speedup 1.0000x reference)
import functools
import jax
import jax.numpy as jnp
from jax import lax
from jax.experimental import pallas as pl
from jax.experimental.pallas import tpu as pltpu
from jax.experimental.pallas import tpu_sc as plsc

SMOOTHING = 0.1
IGNORE_INDEX = 0
LANES = 128


def _make_sc_gather(n_rows, n_classes):
    info = plsc.get_sparse_core_info()
    nc, ns = info.num_cores, info.num_subcores
    nw = nc * ns
    b_per_w = n_rows // nw
    assert n_rows % nw == 0 and b_per_w % 16 == 0
    mesh = plsc.VectorSubcoreMesh(core_axis_name="c", subcore_axis_name="s")

    @functools.partial(
        pl.kernel, mesh=mesh,
        out_type=jax.ShapeDtypeStruct((n_rows,), jnp.float32),
        scratch_types=[
            pltpu.VMEM((b_per_w,), jnp.int32),
            pltpu.VMEM((b_per_w,), jnp.int32),
            pltpu.VMEM((b_per_w,), jnp.float32),
            pltpu.SemaphoreType.DMA,
        ],
    )
    def sc_gather(pred_flat_hbm, tgt_hbm, out_hbm, tgt_v, idx_v, val_v, sem):
        wid = lax.axis_index("s") * nc + lax.axis_index("c")
        base = wid * b_per_w
        pltpu.sync_copy(tgt_hbm.at[pl.ds(base, b_per_w)], tgt_v)
        for j in range(b_per_w // 16):
            row = lax.iota(jnp.int32, 16) + (base + j * 16)
            t = tgt_v[pl.ds(j * 16, 16)]
            idx_v[pl.ds(j * 16, 16)] = row * n_classes + t
        pltpu.async_copy(pred_flat_hbm.at[idx_v], val_v, sem).wait()
        pltpu.sync_copy(val_v, out_hbm.at[pl.ds(base, b_per_w)])

    return sc_gather


def _stats_body(pred_ref, lse_ref, psum_ref, p0_ref, m_ref, s_ref, acc_ref,
                *, n_col_blocks, blk_cols, n_classes):
    cb = pl.program_id(0)
    x = pred_ref[...]
    last = n_col_blocks - 1
    nch = blk_cols // LANES
    rem = n_classes - last * blk_cols
    ones = jnp.ones((blk_cols, 1), jnp.float32)

    def chunk(k):
        return x[:, k * LANES:(k + 1) * LANES]

    @pl.when(cb == 0)
    def _init():
        bm = chunk(0)
        for k in range(1, nch):
            bm = jnp.maximum(bm, chunk(k))
        m_ref[...] = bm
        acc = jnp.exp(chunk(0) - bm)
        for k in range(1, nch):
            acc += jnp.exp(chunk(k) - bm)
        s_ref[...] = acc
        acc_ref[...] = jax.lax.dot_general(
            x, ones, (((1,), (0,)), ((), ())),
            preferred_element_type=jnp.float32)
        p0_ref[...] = x[:, 0:1]

    @pl.when((cb != 0) & (cb != last))
    def _acc():
        bm = chunk(0)
        for k in range(1, nch):
            bm = jnp.maximum(bm, chunk(k))
        m_old = m_ref[...]
        m_new = jnp.maximum(m_old, bm)
        acc = jnp.exp(chunk(0) - m_new)
        for k in range(1, nch):
            acc += jnp.exp(chunk(k) - m_new)
        s_ref[...] = s_ref[...] * jnp.exp(m_old - m_new) + acc
        m_ref[...] = m_new
        acc_ref[...] += jax.lax.dot_general(
            x, ones, (((1,), (0,)), ((), ())),
            preferred_element_type=jnp.float32)

    @pl.when(cb == last)
    def _fin():
        nfull = rem // LANES
        tail = rem - nfull * LANES
        col = jax.lax.broadcasted_iota(jnp.int32, (1, LANES), 1)
        parts = [chunk(k) for k in range(nfull)]
        if tail:
            parts.append(jnp.where(col < tail, chunk(nfull), -jnp.inf))
        bm = parts[0]
        for p in parts[1:]:
            bm = jnp.maximum(bm, p)
        m_old = m_ref[...]
        m_new = jnp.maximum(m_old, bm)
        acc = jnp.exp(parts[0] - m_new)
        for p in parts[1:]:
            acc += jnp.exp(p - m_new)
        s128 = s_ref[...] * jnp.exp(m_old - m_new) + acc

        mrow = jnp.max(m_new, axis=1, keepdims=True)
        srow = jnp.sum(s128 * jnp.exp(m_new - mrow), axis=1, keepdims=True)
        lse_ref[...] = mrow + jnp.log(srow)

        colw = jax.lax.broadcasted_iota(jnp.int32, (1, blk_cols), 1)
        xz = jnp.where(colw < rem, x, 0.0)
        psum_ref[...] = acc_ref[...] + jax.lax.dot_general(
            xz, ones, (((1,), (0,)), ((), ())),
            preferred_element_type=jnp.float32)


def _combine_body(tgt_ref, tval_ref, lse_ref, psum_ref, p0_ref, out_ref, *,
                  n_classes):
    eps = SMOOTHING / (n_classes - 1)
    conf = 1.0 - SMOOTHING
    rows = tgt_ref.shape[0]
    lse = lse_ref[...]
    s_logp = psum_ref[...] - n_classes * lse
    tlp = tval_ref[...] - lse
    zlp = p0_ref[...] - lse
    loss = -(eps * (s_logp - tlp - zlp) + (conf * tlp))
    loss = jnp.where(tgt_ref[...] == IGNORE_INDEX, 0.0, loss)
    out_ref[...] = jnp.sum(loss, axis=0, keepdims=True) / rows


def kernel(pred, target):
    n, c = pred.shape
    tgt32 = target.astype(jnp.int32)
    tval = _make_sc_gather(n, c)(pred.reshape(-1), tgt32)

    blk_cols = 2048
    n_col_blocks = pl.cdiv(c, blk_cols)

    row_spec = pl.BlockSpec((n, 1), lambda cb: (0, 0))
    lse, psum, p0 = pl.pallas_call(
        functools.partial(_stats_body, n_col_blocks=n_col_blocks,
                          blk_cols=blk_cols, n_classes=c),
        grid=(n_col_blocks,),
        in_specs=[pl.BlockSpec((n, blk_cols), lambda cb: (0, cb))],
        out_specs=[row_spec, row_spec, row_spec],
        out_shape=[jax.ShapeDtypeStruct((n, 1), jnp.float32)] * 3,
        scratch_shapes=[
            pltpu.VMEM((n, LANES), jnp.float32),
            pltpu.VMEM((n, LANES), jnp.float32),
            pltpu.VMEM((n, 1), jnp.float32),
        ],
    )(pred)

    spec = pl.BlockSpec((n, 1), lambda: (0, 0))
    out = pl.pallas_call(
        functools.partial(_combine_body, n_classes=c),
        in_specs=[spec] * 5,
        out_specs=pl.BlockSpec((1, 1), lambda: (0, 0)),
        out_shape=jax.ShapeDtypeStruct((1, 1), jnp.float32),
    )(tgt32.reshape(n, 1), tval.reshape(n, 1), lse, psum, p0)
    return out[0, 0]


# speedup vs baseline: 7.2568x; 7.2568x over previous
import functools
import jax
import jax.numpy as jnp
from jax.experimental import pallas as pl
from jax.experimental.pallas import tpu as pltpu

SMOOTHING = 0.1
IGNORE_INDEX = 0


def _loss_body(predt_ref, tgt_ref, out_ref, m_ref, s_ref, tv_ref, ps_ref,
               p0_ref, *, n_blocks, blk, n_classes):
    cb = pl.program_id(0)
    x = predt_ref[...]
    n = x.shape[1]
    nsub = blk // 8
    xr = x.reshape(nsub, 8, n)
    tgt = tgt_ref[...]

    ones = jnp.ones((1, blk), jnp.float32)
    psum_b = jax.lax.dot_general(ones, x, (((1,), (0,)), ((), ())),
                                 preferred_element_type=jnp.float32)

    ci = jax.lax.broadcasted_iota(jnp.int32, (blk, n), 0) + cb * blk
    hit = (ci == tgt).reshape(nsub, 8, n)
    tv_b = jnp.sum(jnp.where(hit, xr, 0.0), axis=0)

    bm = jnp.max(xr, axis=0)

    @pl.when(cb == 0)
    def _init():
        m_ref[...] = bm
        s_ref[...] = jnp.sum(jnp.exp(xr - bm[None]), axis=0)
        ps_ref[...] = psum_b
        tv_ref[...] = tv_b
        p0_ref[...] = x[0:1, :]

    @pl.when(cb != 0)
    def _acc():
        m_old = m_ref[...]
        m_new = jnp.maximum(m_old, bm)
        s_ref[...] = (s_ref[...] * jnp.exp(m_old - m_new)
                      + jnp.sum(jnp.exp(xr - m_new[None]), axis=0))
        m_ref[...] = m_new
        ps_ref[...] += psum_b
        tv_ref[...] += tv_b

    @pl.when(cb == n_blocks - 1)
    def _fin():
        eps = SMOOTHING / (n_classes - 1)
        conf = 1.0 - SMOOTHING
        m8 = m_ref[...]
        mrow = jnp.max(m8, axis=0, keepdims=True)
        srow = jnp.sum(s_ref[...] * jnp.exp(m8 - mrow), axis=0, keepdims=True)
        lse = mrow + jnp.log(srow)
        tval = jnp.sum(tv_ref[...], axis=0, keepdims=True)
        s_logp = ps_ref[...] - n_classes * lse
        tlp = tval - lse
        zlp = p0_ref[...] - lse
        loss = -(eps * (s_logp - tlp - zlp) + conf * tlp)
        loss = jnp.where(tgt == IGNORE_INDEX, 0.0, loss)
        out_ref[...] = jnp.sum(loss, axis=1, keepdims=True) / n


def kernel(pred, target):
    n, c = pred.shape
    predt = pred.T
    tgt2d = target.astype(jnp.int32).reshape(1, n)

    blk = 2000
    if c % blk or blk % 8:
        blk = next(b for b in range(min(c, 2048), 7, -1)
                   if c % b == 0 and b % 8 == 0)
    n_blocks = c // blk

    out = pl.pallas_call(
        functools.partial(_loss_body, n_blocks=n_blocks, blk=blk,
                          n_classes=c),
        grid=(n_blocks,),
        in_specs=[
            pl.BlockSpec((blk, n), lambda cb: (cb, 0)),
            pl.BlockSpec((1, n), lambda cb: (0, 0)),
        ],
        out_specs=pl.BlockSpec((1, 1), lambda cb: (0, 0)),
        out_shape=jax.ShapeDtypeStruct((1, 1), jnp.float32),
        scratch_shapes=[
            pltpu.VMEM((8, n), jnp.float32),
            pltpu.VMEM((8, n), jnp.float32),
            pltpu.VMEM((8, n), jnp.float32),
            pltpu.VMEM((1, n), jnp.float32),
            pltpu.VMEM((1, n), jnp.float32),
        ],
    )(predt, tgt2d)
    return out[0, 0]


# speedup vs baseline: 7.3714x; 1.0158x over previous
import functools
import jax
import jax.numpy as jnp
from jax import lax
from jax.experimental import pallas as pl
from jax.experimental.pallas import tpu as pltpu
from jax.experimental.pallas import tpu_sc as plsc

SMOOTHING = 0.1
IGNORE_INDEX = 0


def _make_sc_tval(n_rows):
    info = plsc.get_sparse_core_info()
    nc, ns = info.num_cores, info.num_subcores
    nw = nc * ns
    b_per_w = n_rows // nw
    assert n_rows % nw == 0 and b_per_w % 16 == 0
    mesh = plsc.VectorSubcoreMesh(core_axis_name="c", subcore_axis_name="s")

    @functools.partial(
        pl.kernel, mesh=mesh,
        out_type=jax.ShapeDtypeStruct((n_rows,), jnp.float32),
        scratch_types=[
            pltpu.VMEM((b_per_w,), jnp.int32),
            pltpu.VMEM((b_per_w, n_rows), jnp.float32),
            pltpu.VMEM((b_per_w,), jnp.float32),
            pltpu.SemaphoreType.DMA,
        ],
    )
    def sc_tval(predt_hbm, tgt_hbm, out_hbm, tgt_v, rows_v, val_v, sem):
        wid = lax.axis_index("s") * nc + lax.axis_index("c")
        base = wid * b_per_w
        pltpu.sync_copy(tgt_hbm.at[pl.ds(base, b_per_w)], tgt_v)
        pltpu.async_copy(predt_hbm.at[tgt_v], rows_v, sem).wait()
        lane = lax.iota(jnp.int32, 16)
        for jj in range(b_per_w // 16):
            acc = jnp.zeros((16,), jnp.float32)
            for l in range(16):
                v = rows_v[jj * 16 + l, pl.ds(base + jj * 16, 16)]
                acc = jnp.where(lane == l, v, acc)
            val_v[pl.ds(jj * 16, 16)] = acc
        pltpu.sync_copy(val_v, out_hbm.at[pl.ds(base, b_per_w)])

    return sc_tval


def _stats_body(predt_ref, part_ref, m_ref, s_ref, ps_ref, p0_ref, *,
                n_blocks, blk, n_classes):
    cb = pl.program_id(0)
    x = predt_ref[...]
    n = x.shape[1]
    nsub = blk // 8
    xr = x.reshape(nsub, 8, n)

    ones = jnp.ones((1, blk), jnp.float32)
    psum_b = jax.lax.dot_general(ones, x, (((1,), (0,)), ((), ())),
                                 preferred_element_type=jnp.float32)
    bm = jnp.max(xr, axis=0)

    @pl.when(cb == 0)
    def _init():
        m_ref[...] = bm
        s_ref[...] = jnp.sum(jnp.exp(xr - bm[None]), axis=0)
        ps_ref[...] = psum_b
        p0_ref[...] = x[0:1, :]

    @pl.when(cb != 0)
    def _acc():
        m_old = m_ref[...]
        m_new = jnp.maximum(m_old, bm)
        s_ref[...] = (s_ref[...] * jnp.exp(m_old - m_new)
                      + jnp.sum(jnp.exp(xr - m_new[None]), axis=0))
        m_ref[...] = m_new
        ps_ref[...] += psum_b

    @pl.when(cb == n_blocks - 1)
    def _fin():
        eps = SMOOTHING / (n_classes - 1)
        conf = 1.0 - SMOOTHING
        m8 = m_ref[...]
        mrow = jnp.max(m8, axis=0, keepdims=True)
        srow = jnp.sum(s_ref[...] * jnp.exp(m8 - mrow), axis=0, keepdims=True)
        lse = mrow + jnp.log(srow)
        s_logp = ps_ref[...] - n_classes * lse
        zlp = p0_ref[...] - lse
        part_ref[...] = -eps * s_logp + eps * zlp - (eps - conf) * lse


def _combine_body(part_ref, tval_ref, tgt_ref, out_ref, *, n_classes):
    eps = SMOOTHING / (n_classes - 1)
    conf = 1.0 - SMOOTHING
    n = tgt_ref.shape[1]
    loss = part_ref[...] + (eps - conf) * tval_ref[...]
    loss = jnp.where(tgt_ref[...] == IGNORE_INDEX, 0.0, loss)
    out_ref[...] = jnp.sum(loss, axis=1, keepdims=True) / n


def kernel(pred, target):
    n, c = pred.shape
    predt = pred.T
    tgt32 = target.astype(jnp.int32)
    tval = _make_sc_tval(n)(predt, tgt32)

    blk = 2000
    if c % blk or blk % 8:
        blk = next(b for b in range(min(c, 2048), 7, -1)
                   if c % b == 0 and b % 8 == 0)
    n_blocks = c // blk

    part = pl.pallas_call(
        functools.partial(_stats_body, n_blocks=n_blocks, blk=blk,
                          n_classes=c),
        grid=(n_blocks,),
        in_specs=[pl.BlockSpec((blk, n), lambda cb: (cb, 0))],
        out_specs=pl.BlockSpec((1, n), lambda cb: (0, 0)),
        out_shape=jax.ShapeDtypeStruct((1, n), jnp.float32),
        scratch_shapes=[
            pltpu.VMEM((8, n), jnp.float32),
            pltpu.VMEM((8, n), jnp.float32),
            pltpu.VMEM((1, n), jnp.float32),
            pltpu.VMEM((1, n), jnp.float32),
        ],
    )(predt)

    spec = pl.BlockSpec((1, n), lambda: (0, 0))
    out = pl.pallas_call(
        functools.partial(_combine_body, n_classes=c),
        in_specs=[spec] * 3,
        out_specs=pl.BlockSpec((1, 1), lambda: (0, 0)),
        out_shape=jax.ShapeDtypeStruct((1, 1), jnp.float32),
    )(part, tval.reshape(1, n), tgt32.reshape(1, n))
    return out[0, 0]


# speedup vs baseline: 7.5503x; 1.0243x over previous
import functools
import jax
import jax.numpy as jnp
from jax import lax
from jax.experimental import pallas as pl
from jax.experimental.pallas import tpu as pltpu
from jax.experimental.pallas import tpu_sc as plsc

SMOOTHING = 0.1
IGNORE_INDEX = 0


def _make_sc_tval(n_rows):
    info = plsc.get_sparse_core_info()
    nc, ns = info.num_cores, info.num_subcores
    nw = nc * ns
    b_per_w = n_rows // nw
    assert n_rows % nw == 0 and b_per_w % 16 == 0
    mesh = plsc.VectorSubcoreMesh(core_axis_name="c", subcore_axis_name="s")

    @functools.partial(
        pl.kernel, mesh=mesh,
        out_type=jax.ShapeDtypeStruct((n_rows,), jnp.float32),
        scratch_types=[
            pltpu.VMEM((b_per_w,), jnp.int32),
            pltpu.VMEM((b_per_w, n_rows), jnp.float32),
            pltpu.VMEM((b_per_w,), jnp.float32),
            pltpu.SemaphoreType.DMA,
        ],
    )
    def sc_tval(predt_hbm, tgt_hbm, out_hbm, tgt_v, rows_v, val_v, sem):
        wid = lax.axis_index("s") * nc + lax.axis_index("c")
        base = wid * b_per_w
        pltpu.sync_copy(tgt_hbm.at[pl.ds(base, b_per_w)], tgt_v)
        pltpu.async_copy(predt_hbm.at[tgt_v], rows_v, sem).wait()
        lane = lax.iota(jnp.int32, 16)
        for jj in range(b_per_w // 16):
            acc = jnp.zeros((16,), jnp.float32)
            for l in range(16):
                v = rows_v[jj * 16 + l, pl.ds(base + jj * 16, 16)]
                acc = jnp.where(lane == l, v, acc)
            val_v[pl.ds(jj * 16, 16)] = acc
        pltpu.sync_copy(val_v, out_hbm.at[pl.ds(base, b_per_w)])

    return sc_tval


def _stats_body(predt_ref, part_ref, m_ref, s_ref, ps_ref, p0_ref, *,
                n_blocks, blk, n_classes):
    cb = pl.program_id(0)
    x = predt_ref[...]
    n = x.shape[1]
    nsub = blk // 8
    xr = x.reshape(nsub, 8, n)

    ones = jnp.ones((1, blk), jnp.float32)
    psum_b = jax.lax.dot_general(ones, x, (((1,), (0,)), ((), ())),
                                 preferred_element_type=jnp.float32)
    bm = jnp.max(xr, axis=0)

    @pl.when(cb == 0)
    def _init():
        m_ref[...] = bm
        s_ref[...] = jnp.sum(jnp.exp(xr - bm[None]), axis=0)
        ps_ref[...] = psum_b
        p0_ref[...] = x[0:1, :]

    @pl.when(cb != 0)
    def _acc():
        m_old = m_ref[...]
        m_new = jnp.maximum(m_old, bm)
        s_ref[...] = (s_ref[...] * jnp.exp(m_old - m_new)
                      + jnp.sum(jnp.exp(xr - m_new[None]), axis=0))
        m_ref[...] = m_new
        ps_ref[...] += psum_b

    @pl.when(cb == n_blocks - 1)
    def _fin():
        eps = SMOOTHING / (n_classes - 1)
        conf = 1.0 - SMOOTHING
        m8 = m_ref[...]
        mrow = jnp.max(m8, axis=0, keepdims=True)
        srow = jnp.sum(s_ref[...] * jnp.exp(m8 - mrow), axis=0, keepdims=True)
        lse = mrow + jnp.log(srow)
        s_logp = ps_ref[...] - n_classes * lse
        zlp = p0_ref[...] - lse
        part_ref[...] = -eps * s_logp + eps * zlp - (eps - conf) * lse


def _combine_body(part_ref, tval_ref, tgt_ref, out_ref, *, n_classes):
    eps = SMOOTHING / (n_classes - 1)
    conf = 1.0 - SMOOTHING
    n = tgt_ref.shape[1]
    loss = part_ref[...] + (eps - conf) * tval_ref[...]
    loss = jnp.where(tgt_ref[...] == IGNORE_INDEX, 0.0, loss)
    out_ref[...] = jnp.sum(loss, axis=1, keepdims=True) / n


def kernel(pred, target):
    n, c = pred.shape
    predt = pred.T
    tgt32 = target.astype(jnp.int32)
    tval = _make_sc_tval(n)(predt, tgt32)

    blk = 4000
    if c % blk or blk % 8:
        blk = next(b for b in range(min(c, 2048), 7, -1)
                   if c % b == 0 and b % 8 == 0)
    n_blocks = c // blk

    part = pl.pallas_call(
        functools.partial(_stats_body, n_blocks=n_blocks, blk=blk,
                          n_classes=c),
        grid=(n_blocks,),
        in_specs=[pl.BlockSpec((blk, n), lambda cb: (cb, 0))],
        out_specs=pl.BlockSpec((1, n), lambda cb: (0, 0)),
        out_shape=jax.ShapeDtypeStruct((1, n), jnp.float32),
        scratch_shapes=[
            pltpu.VMEM((8, n), jnp.float32),
            pltpu.VMEM((8, n), jnp.float32),
            pltpu.VMEM((1, n), jnp.float32),
            pltpu.VMEM((1, n), jnp.float32),
        ],
    )(predt)

    spec = pl.BlockSpec((1, n), lambda: (0, 0))
    out = pl.pallas_call(
        functools.partial(_combine_body, n_classes=c),
        in_specs=[spec] * 3,
        out_specs=pl.BlockSpec((1, 1), lambda: (0, 0)),
        out_shape=jax.ShapeDtypeStruct((1, 1), jnp.float32),
    )(part, tval.reshape(1, n), tgt32.reshape(1, n))
    return out[0, 0]
